# Initial kernel scaffold; baseline (speedup 1.0000x reference)
import jax
import jax.numpy as jnp
from jax.experimental import pallas as pl


def kernel(x, edge_index, W_l, W_r, b):
    raise NotImplementedError("write your pallas kernel here")


# speedup vs baseline: 3.7125x; 3.7125x over previous
import functools

import jax
import jax.numpy as jnp
from jax import lax
from jax.experimental import pallas as pl
from jax.experimental.pallas import tpu as pltpu
from jax.experimental.pallas import tpu_sc as plsc

N = 10000
E = 160000
D = 256
DH = 128
N_PAD = 10240
TRASH = N
NSUB = 16
CHUNK = 128
CH_PER_SUB = 79
EDGES_PER_SUB = CH_PER_SUB * CHUNK
E_PAD = NSUB * EDGES_PER_SUB
ROWS_PER_SUB = N_PAD // NSUB

_MESH = plsc.VectorSubcoreMesh(core_axis_name="c", subcore_axis_name="s")


@functools.partial(
    pl.kernel,
    out_type=(jax.ShapeDtypeStruct((2, N_PAD, DH), jnp.float32),
              jax.ShapeDtypeStruct((N_PAD,), jnp.float32)),
    mesh=_MESH,
    compiler_params=pltpu.CompilerParams(needs_layout_passes=False),
    scratch_types=[
        pltpu.VMEM((CHUNK,), jnp.int32),
        pltpu.VMEM((CHUNK,), jnp.int32),
        pltpu.VMEM((CHUNK, DH), jnp.float32),
        pltpu.VMEM((N_PAD,), jnp.float32),
        pltpu.VMEM((ROWS_PER_SUB,), jnp.float32),
        pltpu.VMEM((ROWS_PER_SUB,), jnp.float32),
        pltpu.VMEM_SHARED((N_PAD, DH), jnp.float32),
        pltpu.VMEM_SHARED((NSUB, N_PAD), jnp.float32),
    ],
)
def _segsum(tables_hbm, src2_hbm, dst_hbm, out_hbm, cnt_hbm,
            src_v, dst_v, rows_v, cnt_v, cacc_v, cred_v, acc_sh, cstage_sh):
    ci = lax.axis_index("c")
    si = lax.axis_index("s")
    zv = jnp.zeros((16,), jnp.float32)
    ones16 = jnp.ones((16,), jnp.float32)

    @pl.loop(0, CHUNK)
    def _(r):
        @pl.loop(0, DH // 16)
        def _(cc):
            rows_v[r, pl.ds(cc * 16, 16)] = zv

    @pl.loop(0, ROWS_PER_SUB // CHUNK)
    def _(k):
        pltpu.sync_copy(rows_v, acc_sh.at[pl.ds(si * ROWS_PER_SUB + k * CHUNK, CHUNK)])

    @pl.loop(0, N_PAD // 16)
    def _(k):
        cnt_v[pl.ds(k * 16, 16)] = zv

    plsc.subcore_barrier()

    base = si * EDGES_PER_SUB

    @pl.loop(0, CH_PER_SUB)
    def _(c):
        off = base + c * CHUNK
        pltpu.sync_copy(src2_hbm.at[ci, pl.ds(off, CHUNK)], src_v)
        pltpu.sync_copy(dst_hbm.at[pl.ds(off, CHUNK)], dst_v)
        pltpu.sync_copy(tables_hbm.at[src_v], rows_v)
        pltpu.sync_copy(rows_v, acc_sh.at[dst_v], add=True)

        @pl.loop(0, CHUNK // 16)
        def _(j):
            idx16 = dst_v[pl.ds(j * 16, 16)]
            plsc.addupdate_scatter(cnt_v, [idx16], ones16)

    plsc.subcore_barrier()
    pltpu.sync_copy(acc_sh.at[pl.ds(si * ROWS_PER_SUB, ROWS_PER_SUB)],
                    out_hbm.at[ci, pl.ds(si * ROWS_PER_SUB, ROWS_PER_SUB)])

    @pl.when(ci == 0)
    def _():
        pltpu.sync_copy(cnt_v, cstage_sh.at[si])
        plsc.subcore_barrier()

        @pl.loop(0, ROWS_PER_SUB // 16)
        def _(k):
            cacc_v[pl.ds(k * 16, 16)] = zv

        @pl.loop(0, NSUB)
        def _(t):
            pltpu.sync_copy(cstage_sh.at[t, pl.ds(si * ROWS_PER_SUB, ROWS_PER_SUB)],
                            cred_v)

            @pl.loop(0, ROWS_PER_SUB // 16)
            def _(k):
                s = pl.ds(k * 16, 16)
                cacc_v[s] = cacc_v[s] + cred_v[s]

        pltpu.sync_copy(cacc_v, cnt_hbm.at[pl.ds(si * ROWS_PER_SUB, ROWS_PER_SUB)])


BLK = 1000


def _tc_body(sums_ref, cnt_ref, x_ref, wl_ref, wr_ref, b_ref, out_ref):
    s0 = sums_ref[0]
    s1 = sums_ref[1]
    inv = 1.0 / jnp.maximum(cnt_ref[...], 1.0)
    agg = jnp.concatenate([s0, s1], axis=1) * inv
    acc = jnp.dot(agg, wl_ref[...], preferred_element_type=jnp.float32)
    acc = acc + jnp.dot(x_ref[...], wr_ref[...], preferred_element_type=jnp.float32)
    out_ref[...] = acc + b_ref[...]


def kernel(x, edge_index, W_l, W_r, b):
    x = x.astype(jnp.float32)
    src = edge_index[0].astype(jnp.int32)
    dst = edge_index[1].astype(jnp.int32)

    rpad = jnp.zeros((N_PAD - N, DH), jnp.float32)
    tables = jnp.concatenate([x[:, :DH], rpad, x[:, DH:], rpad], axis=0)

    epad = jnp.full((E_PAD - E,), TRASH, jnp.int32)
    src_p = jnp.concatenate([src, epad])
    dst_p = jnp.concatenate([dst, epad])
    src2 = jnp.stack([src_p, src_p + N_PAD])

    sums, counts = _segsum(tables, src2, dst_p)

    return pl.pallas_call(
        _tc_body,
        grid=(N // BLK,),
        in_specs=[
            pl.BlockSpec((2, BLK, DH), lambda i: (0, i, 0)),
            pl.BlockSpec((BLK, 1), lambda i: (i, 0)),
            pl.BlockSpec((BLK, D), lambda i: (i, 0)),
            pl.BlockSpec((D, D), lambda i: (0, 0)),
            pl.BlockSpec((D, D), lambda i: (0, 0)),
            pl.BlockSpec((1, D), lambda i: (0, 0)),
        ],
        out_specs=pl.BlockSpec((BLK, D), lambda i: (i, 0)),
        out_shape=jax.ShapeDtypeStruct((N, D), jnp.float32),
    )(sums, counts.reshape(N_PAD, 1), x, W_l, W_r, b.reshape(1, D))
